# baseline (speedup 1.0000x reference)
import jax
import jax.numpy as jnp
from jax.experimental import pallas as pl

N_CLS = 1000
PROMPT_LEN = 16
EMBED_DIM = 768
CTX_LEN = 77
SUF_LEN = CTX_LEN - 1 - PROMPT_LEN


def _concat_body(pre_ref, prm_ref, suf_ref, out_ref):
    out_ref[...] = jnp.concatenate(
        [pre_ref[...], prm_ref[...], suf_ref[...]], axis=1
    )


def kernel(new_prompt_tokens, token_prefix, token_suffix, tokenized_prompts):
    C = 8
    prompts = pl.pallas_call(
        _concat_body,
        grid=(N_CLS // C,),
        in_specs=[
            pl.BlockSpec((C, 1, EMBED_DIM), lambda i: (i, 0, 0)),
            pl.BlockSpec((C, PROMPT_LEN, EMBED_DIM), lambda i: (i, 0, 0)),
            pl.BlockSpec((C, SUF_LEN, EMBED_DIM), lambda i: (i, 0, 0)),
        ],
        out_specs=pl.BlockSpec((C, CTX_LEN, EMBED_DIM), lambda i: (i, 0, 0)),
        out_shape=jax.ShapeDtypeStruct((N_CLS, CTX_LEN, EMBED_DIM), jnp.float32),
    )(token_prefix, new_prompt_tokens, token_suffix)
    return (tokenized_prompts, prompts)


# speedup vs baseline: 1.0440x; 1.0440x over previous
import jax
import jax.numpy as jnp
from jax.experimental import pallas as pl

N_CLS = 1000
PROMPT_LEN = 16
EMBED_DIM = 768
CTX_LEN = 77
SUF_LEN = CTX_LEN - 1 - PROMPT_LEN


def _concat_body(pre_ref, prm_ref, suf_ref, out_ref):
    out_ref[...] = jnp.concatenate(
        [pre_ref[...], prm_ref[...], suf_ref[...]], axis=1
    )


def kernel(new_prompt_tokens, token_prefix, token_suffix, tokenized_prompts):
    C = 40
    prompts = pl.pallas_call(
        _concat_body,
        grid=(N_CLS // C,),
        in_specs=[
            pl.BlockSpec((C, 1, EMBED_DIM), lambda i: (i, 0, 0)),
            pl.BlockSpec((C, PROMPT_LEN, EMBED_DIM), lambda i: (i, 0, 0)),
            pl.BlockSpec((C, SUF_LEN, EMBED_DIM), lambda i: (i, 0, 0)),
        ],
        out_specs=pl.BlockSpec((C, CTX_LEN, EMBED_DIM), lambda i: (i, 0, 0)),
        out_shape=jax.ShapeDtypeStruct((N_CLS, CTX_LEN, EMBED_DIM), jnp.float32),
    )(token_prefix, new_prompt_tokens, token_suffix)
    return (tokenized_prompts, prompts)
